# baseline (speedup 1.0000x reference)
import functools

import jax
import jax.numpy as jnp
from jax import lax
from jax.experimental import pallas as pl
from jax.experimental.pallas import tpu as pltpu
from jax.experimental.pallas import tpu_sc as plsc

_NUM_CLASSES = 100000
_NUM_FEATURES = 64
_BATCH = 1024
_BLK = 1000
_NBLK = (_NUM_CLASSES // 2) // _BLK
_INV_TEMP = 20.0


def _gather_target_rows(bank2, idx):
    info = plsc.get_sparse_core_info()
    nw = info.num_cores * info.num_subcores
    b_per_w = _BATCH // nw
    mesh = plsc.VectorSubcoreMesh(core_axis_name="c", subcore_axis_name="s")

    @functools.partial(
        pl.kernel, mesh=mesh,
        out_type=jax.ShapeDtypeStruct((_BATCH, 2 * _NUM_FEATURES), jnp.float32),
        scratch_types=[
            pltpu.VMEM((b_per_w,), jnp.int32),
            pltpu.VMEM((b_per_w, 2 * _NUM_FEATURES), jnp.float32),
            pltpu.SemaphoreType.DMA,
        ],
    )
    def k(table_hbm, idx_hbm, out_hbm, idx_v, rows_v, sem):
        wid = lax.axis_index("s") * info.num_cores + lax.axis_index("c")
        base = wid * b_per_w
        pltpu.sync_copy(idx_hbm.at[pl.ds(base, b_per_w)], idx_v)
        pltpu.async_copy(table_hbm.at[idx_v], rows_v, sem).wait()
        pltpu.sync_copy(rows_v, out_hbm.at[pl.ds(base, b_per_w)])

    return k(bank2, idx)


def _loss_kernel(x_ref, bank_ref, rows_ref, par_ref, out_ref, ni_ref, m_ref, s_ref):
    j = pl.program_id(0)

    @pl.when(j == 0)
    def _init():
        x = x_ref[...]
        nrm = jnp.sqrt(jnp.sum(x * x, axis=1, keepdims=True))
        ni_ref[...] = (x * (_INV_TEMP / jnp.maximum(nrm, 1e-12))).astype(jnp.bfloat16)
        m_ref[...] = jnp.full((1, _BATCH), -1e30, jnp.float32)
        s_ref[...] = jnp.zeros((1, _BATCH), jnp.float32)

    bank = bank_ref[...].astype(jnp.bfloat16)
    ni = ni_ref[...]
    dn = (((1,), (1,)), ((), ()))
    le = lax.dot_general(bank[:, :_NUM_FEATURES], ni, dn,
                         preferred_element_type=jnp.float32)
    lo = lax.dot_general(bank[:, _NUM_FEATURES:], ni, dn,
                         preferred_element_type=jnp.float32)
    m_old = m_ref[...]
    bm = jnp.max(jnp.maximum(le, lo), axis=0, keepdims=True)
    m_new = jnp.maximum(m_old, bm)
    p = jnp.exp(le - m_new) + jnp.exp(lo - m_new)
    s_ref[...] = s_ref[...] * jnp.exp(m_old - m_new) + jnp.sum(p, axis=0, keepdims=True)
    m_ref[...] = m_new

    @pl.when(j == _NBLK - 1)
    def _fin():
        lse_sum = jnp.sum(m_ref[...] + jnp.log(s_ref[...]))
        odd = par_ref[...] != 0
        row = jnp.where(odd, rows_ref[:, _NUM_FEATURES:], rows_ref[:, :_NUM_FEATURES])
        tgt_sum = jnp.sum(row * ni_ref[...].astype(jnp.float32))
        out_ref[0, 0] = (lse_sum - tgt_sum) * (1.0 / _BATCH)


def kernel(inputs, targets, features_bank):
    tgt = targets.astype(jnp.int32)
    bank2 = features_bank.reshape(_NUM_CLASSES // 2, 2 * _NUM_FEATURES)
    rows = _gather_target_rows(bank2, tgt // 2)
    loss = pl.pallas_call(
        _loss_kernel,
        grid=(_NBLK,),
        in_specs=[
            pl.BlockSpec((_BATCH, _NUM_FEATURES), lambda j: (0, 0)),
            pl.BlockSpec((_BLK, 2 * _NUM_FEATURES), lambda j: (j, 0)),
            pl.BlockSpec((_BATCH, 2 * _NUM_FEATURES), lambda j: (0, 0)),
            pl.BlockSpec((_BATCH, 1), lambda j: (0, 0)),
        ],
        out_specs=pl.BlockSpec(memory_space=pltpu.SMEM),
        out_shape=jax.ShapeDtypeStruct((1, 1), jnp.float32),
        scratch_shapes=[
            pltpu.VMEM((_BATCH, _NUM_FEATURES), jnp.bfloat16),
            pltpu.VMEM((1, _BATCH), jnp.float32),
            pltpu.VMEM((1, _BATCH), jnp.float32),
        ],
    )(inputs, bank2, rows, (tgt % 2).reshape(_BATCH, 1))
    return loss[0, 0]


# speedup vs baseline: 1.1242x; 1.1242x over previous
import functools

import jax
import jax.numpy as jnp
from jax import lax
from jax.experimental import pallas as pl
from jax.experimental.pallas import tpu as pltpu
from jax.experimental.pallas import tpu_sc as plsc

_NUM_CLASSES = 100000
_NUM_FEATURES = 64
_BATCH = 1024
_BLK = 1000
_CLS = 2 * _BLK
_NBLK = (_NUM_CLASSES // 2) // _BLK
_LN2 = 0.6931471805599453
_SCALE = 20.0 * 1.4426950408889634


def _gather_target_rows(bank2, idx):
    info = plsc.get_sparse_core_info()
    nw = info.num_cores * info.num_subcores
    b_per_w = _BATCH // nw
    mesh = plsc.VectorSubcoreMesh(core_axis_name="c", subcore_axis_name="s")

    @functools.partial(
        pl.kernel, mesh=mesh,
        out_type=jax.ShapeDtypeStruct((_BATCH, 2 * _NUM_FEATURES), jnp.float32),
        scratch_types=[
            pltpu.VMEM((b_per_w,), jnp.int32),
            pltpu.VMEM((b_per_w, 2 * _NUM_FEATURES), jnp.float32),
            pltpu.SemaphoreType.DMA,
        ],
    )
    def k(table_hbm, idx_hbm, out_hbm, idx_v, rows_v, sem):
        wid = lax.axis_index("s") * info.num_cores + lax.axis_index("c")
        base = wid * b_per_w
        pltpu.sync_copy(idx_hbm.at[pl.ds(base, b_per_w)], idx_v)
        pltpu.async_copy(table_hbm.at[idx_v], rows_v, sem).wait()
        pltpu.sync_copy(rows_v, out_hbm.at[pl.ds(base, b_per_w)])

    return k(bank2, idx)


def _compute_logits(bank_ref, ni, lbuf_ref):
    bank = bank_ref[...].astype(jnp.bfloat16)
    dn = (((1,), (1,)), ((), ()))
    lbuf_ref[:_BLK, :] = lax.dot_general(
        bank[:, :_NUM_FEATURES], ni, dn, preferred_element_type=jnp.float32)
    lbuf_ref[_BLK:, :] = lax.dot_general(
        bank[:, _NUM_FEATURES:], ni, dn, preferred_element_type=jnp.float32)


def _reduce_block(lbuf_ref, ones, m_ref, s_ref):
    l = lbuf_ref[...]
    m_old = m_ref[...]
    m_new = jnp.maximum(m_old, jnp.max(l, axis=0, keepdims=True))
    y = (l - m_new).astype(jnp.bfloat16)
    p = jnp.exp2(y)
    psum = lax.dot_general(ones, p, (((1,), (0,)), ((), ())),
                           preferred_element_type=jnp.float32)
    s_ref[...] = s_ref[...] * jnp.exp2(m_old - m_new) + psum
    m_ref[...] = m_new


def _loss_kernel(x_ref, bank_ref, rows_ref, par_ref, out_ref,
                 ni_ref, m_ref, s_ref, l0_ref, l1_ref):
    j = pl.program_id(0)
    ones = jnp.ones((1, _CLS), jnp.bfloat16)

    @pl.when(j == 0)
    def _init():
        x = x_ref[...]
        nrm = jnp.sqrt(jnp.sum(x * x, axis=1, keepdims=True))
        ni_ref[...] = (x * (_SCALE / jnp.maximum(nrm, 1e-12))).astype(jnp.bfloat16)
        m_ref[...] = jnp.full((1, _BATCH), -1e30, jnp.float32)
        s_ref[...] = jnp.zeros((1, _BATCH), jnp.float32)
        l1_ref[...] = jnp.full((_CLS, _BATCH), -3e30, jnp.float32)

    @pl.when(j % 2 == 0)
    def _even():
        _compute_logits(bank_ref, ni_ref[...], l0_ref)
        _reduce_block(l1_ref, ones, m_ref, s_ref)

    @pl.when(j % 2 == 1)
    def _odd():
        _compute_logits(bank_ref, ni_ref[...], l1_ref)
        _reduce_block(l0_ref, ones, m_ref, s_ref)

    @pl.when(j == _NBLK)
    def _fin():
        lse_sum = jnp.sum(m_ref[...] + jnp.log2(s_ref[...]))
        odd = par_ref[...] != 0
        row = jnp.where(odd, rows_ref[:, _NUM_FEATURES:], rows_ref[:, :_NUM_FEATURES])
        tgt_sum = jnp.sum(row * ni_ref[...].astype(jnp.float32))
        out_ref[0, 0] = (lse_sum - tgt_sum) * (_LN2 / _BATCH)


def kernel(inputs, targets, features_bank):
    tgt = targets.astype(jnp.int32)
    bank2 = features_bank.reshape(_NUM_CLASSES // 2, 2 * _NUM_FEATURES)
    rows = _gather_target_rows(bank2, tgt // 2)
    loss = pl.pallas_call(
        _loss_kernel,
        grid=(_NBLK + 1,),
        in_specs=[
            pl.BlockSpec((_BATCH, _NUM_FEATURES), lambda j: (0, 0)),
            pl.BlockSpec((_BLK, 2 * _NUM_FEATURES),
                         lambda j: (jnp.minimum(j, _NBLK - 1), 0)),
            pl.BlockSpec((_BATCH, 2 * _NUM_FEATURES), lambda j: (0, 0)),
            pl.BlockSpec((_BATCH, 1), lambda j: (0, 0)),
        ],
        out_specs=pl.BlockSpec(memory_space=pltpu.SMEM),
        out_shape=jax.ShapeDtypeStruct((1, 1), jnp.float32),
        scratch_shapes=[
            pltpu.VMEM((_BATCH, _NUM_FEATURES), jnp.bfloat16),
            pltpu.VMEM((1, _BATCH), jnp.float32),
            pltpu.VMEM((1, _BATCH), jnp.float32),
            pltpu.VMEM((_CLS, _BATCH), jnp.float32),
            pltpu.VMEM((_CLS, _BATCH), jnp.float32),
        ],
    )(inputs, bank2, rows, (tgt % 2).reshape(_BATCH, 1))
    return loss[0, 0]
